# Initial kernel scaffold; baseline (speedup 1.0000x reference)
import jax
import jax.numpy as jnp
from jax.experimental import pallas as pl


def kernel(hidden_states, gate_w, gate_proj_w, up_proj_w, down_proj_w):
    raise NotImplementedError("write your pallas kernel here")


# speedup vs baseline: 2.5565x; 2.5565x over previous
import functools

import jax
import jax.numpy as jnp
from jax import lax
from jax.experimental import pallas as pl
from jax.experimental.pallas import tpu as pltpu

C = 256
TILE = 64
NT = C // TILE


def _router_kernel(x_ref, gw_ref, logits_ref, a1_ref, a2_ref, w1_ref, w2_ref):
    x = x_ref[...]
    gw = gw_ref[...]
    logits = lax.dot_general(x, gw, (((1,), (1,)), ((), ())),
                             preferred_element_type=jnp.float32)
    logits_ref[...] = logits
    e = gw.shape[0]
    col = lax.broadcasted_iota(jnp.int32, logits.shape, 1)
    m1 = jnp.max(logits, axis=1, keepdims=True)
    a1 = jnp.min(jnp.where(logits == m1, col, e), axis=1, keepdims=True)
    l2 = jnp.where(col == a1, -jnp.inf, logits)
    m2 = jnp.max(l2, axis=1, keepdims=True)
    a2 = jnp.min(jnp.where(l2 == m2, col, e), axis=1, keepdims=True)
    s2 = jnp.exp(m2 - m1)
    w1 = 1.0 / (1.0 + s2)
    a1_ref[...] = a1
    a2_ref[...] = a2
    w1_ref[...] = w1
    w2_ref[...] = s2 * w1


def _route(x, gate_w):
    t, _ = x.shape
    e = gate_w.shape[0]
    f32 = jnp.float32
    return pl.pallas_call(
        _router_kernel,
        out_shape=(
            jax.ShapeDtypeStruct((t, e), f32),
            jax.ShapeDtypeStruct((t, 1), jnp.int32),
            jax.ShapeDtypeStruct((t, 1), jnp.int32),
            jax.ShapeDtypeStruct((t, 1), f32),
            jax.ShapeDtypeStruct((t, 1), f32),
        ),
    )(x, gate_w)


def _dispatch_jnp(a1, a2, w1, w2, num_experts):
    t = a1.shape[0]
    hot1 = jax.nn.one_hot(a1, num_experts, dtype=jnp.int32)
    hot2 = jax.nn.one_hot(a2, num_experts, dtype=jnp.int32)
    hit = hot1 + hot2
    pos = jnp.cumsum(hit, axis=0) - hit
    cnt = jnp.minimum(hit.sum(axis=0), C).astype(jnp.int32)
    p1 = jnp.take_along_axis(pos, a1[:, None], 1)[:, 0]
    p2 = jnp.take_along_axis(pos, a2[:, None], 1)[:, 0]
    tok = jnp.arange(t, dtype=jnp.int32)
    idx0 = jnp.zeros((num_experts + 1, C), jnp.int32)
    wt0 = jnp.zeros((num_experts + 1, C), jnp.float32)
    row1 = jnp.where(p1 < C, a1, num_experts)
    row2 = jnp.where(p2 < C, a2, num_experts)
    p1c = jnp.minimum(p1, C - 1)
    p2c = jnp.minimum(p2, C - 1)
    idx0 = idx0.at[row1, p1c].set(tok).at[row2, p2c].set(tok)
    wt0 = wt0.at[row1, p1c].set(w1).at[row2, p2c].set(w2)
    return idx0[:num_experts], wt0[:num_experts], cnt


def _ffn_kernel(idx_ref, cnt_ref, x_ref, gw_ref, uw_ref, dw_ref, wt_ref,
                out_ref, xe_ref, d_ref):
    e = pl.program_id(0)
    j = pl.program_id(1)

    @pl.when((e == 0) & (j == 0))
    def _init():
        out_ref[...] = jnp.zeros_like(out_ref)

    ne = cnt_ref[e]
    base = j * TILE

    @pl.when(base < ne)
    def _tile():
        def gather(r, carry):
            tok = idx_ref[e, base + r]
            xe_ref[pl.ds(r, 1), :] = x_ref[pl.ds(tok, 1), :]
            return carry
        lax.fori_loop(0, TILE, gather, 0)
        xe = xe_ref[...]
        g = lax.dot_general(xe, gw_ref[0], (((1,), (1,)), ((), ())),
                            preferred_element_type=jnp.float32)
        u = lax.dot_general(xe, uw_ref[0], (((1,), (1,)), ((), ())),
                            preferred_element_type=jnp.float32)
        h = (g * jax.nn.sigmoid(g)) * u
        d = lax.dot_general(h, dw_ref[0], (((1,), (1,)), ((), ())),
                            preferred_element_type=jnp.float32)
        d_ref[...] = d * wt_ref[0]

        def scat(r, carry):
            tok = idx_ref[e, base + r]
            out_ref[pl.ds(tok, 1), :] = (out_ref[pl.ds(tok, 1), :]
                                         + d_ref[pl.ds(r, 1), :])
            return carry
        lax.fori_loop(0, TILE, scat, 0)


def _ffn(idx, cnt, x, gate_proj_w, up_proj_w, down_proj_w, wt):
    t, h = x.shape
    e, i, _ = gate_proj_w.shape
    wt3 = wt.reshape(e * NT, TILE, 1)
    grid_spec = pltpu.PrefetchScalarGridSpec(
        num_scalar_prefetch=2,
        grid=(e, NT),
        in_specs=[
            pl.BlockSpec((t, h), lambda ei, ji, idx_r, cnt_r: (0, 0)),
            pl.BlockSpec((1, i, h), lambda ei, ji, idx_r, cnt_r: (ei, 0, 0)),
            pl.BlockSpec((1, i, h), lambda ei, ji, idx_r, cnt_r: (ei, 0, 0)),
            pl.BlockSpec((1, h, i), lambda ei, ji, idx_r, cnt_r: (ei, 0, 0)),
            pl.BlockSpec((1, TILE, 1),
                         lambda ei, ji, idx_r, cnt_r: (ei * NT + ji, 0, 0)),
        ],
        out_specs=pl.BlockSpec((t, h), lambda ei, ji, idx_r, cnt_r: (0, 0)),
        scratch_shapes=[
            pltpu.VMEM((TILE, h), jnp.float32),
            pltpu.VMEM((TILE, h), jnp.float32),
        ],
    )
    return pl.pallas_call(
        _ffn_kernel,
        grid_spec=grid_spec,
        out_shape=jax.ShapeDtypeStruct((t, h), jnp.float32),
        compiler_params=pltpu.CompilerParams(
            dimension_semantics=("arbitrary", "arbitrary")),
    )(idx, cnt, x, gate_proj_w, up_proj_w, down_proj_w, wt3)


def kernel(hidden_states, gate_w, gate_proj_w, up_proj_w, down_proj_w):
    b, s, h = hidden_states.shape
    e = gate_w.shape[0]
    x = hidden_states.reshape(-1, h)
    logits, a1, a2, w1, w2 = _route(x, gate_w)
    a1 = a1[:, 0]
    a2 = a2[:, 0]
    w1 = w1[:, 0]
    w2 = w2[:, 0]
    idx, wt, cnt = _dispatch_jnp(a1, a2, w1, w2, e)
    final = _ffn(idx, cnt, x, gate_proj_w, up_proj_w, down_proj_w, wt)
    return final.reshape(b, s, h), logits


# speedup vs baseline: 2.8772x; 1.1254x over previous
import functools

import jax
import jax.numpy as jnp
from jax import lax
from jax.experimental import pallas as pl
from jax.experimental.pallas import tpu as pltpu
from jax.experimental.pallas import tpu_sc as plsc

C = 256
TILE = 64
NT = C // TILE
TB = 256


def _router_kernel(x_ref, gw_ref, logits_ref, a1_ref, a2_ref, p1_ref, p2_ref,
                   s1_ref, s2_ref, w1_ref, w2_ref, cnt_ref):
    x = x_ref[...]
    gw = gw_ref[...]
    logits = lax.dot_general(x, gw, (((1,), (1,)), ((), ())),
                             preferred_element_type=jnp.float32)
    logits_ref[...] = logits
    e = gw.shape[0]
    t = logits.shape[0]
    col = lax.broadcasted_iota(jnp.int32, logits.shape, 1)
    m1 = jnp.max(logits, axis=1, keepdims=True)
    a1 = jnp.min(jnp.where(logits == m1, col, e), axis=1, keepdims=True)
    l2 = jnp.where(col == a1, -jnp.inf, logits)
    m2 = jnp.max(l2, axis=1, keepdims=True)
    a2 = jnp.min(jnp.where(l2 == m2, col, e), axis=1, keepdims=True)
    s2 = jnp.exp(m2 - m1)
    w1 = 1.0 / (1.0 + s2)
    w2 = s2 * w1
    a1_ref[...] = a1
    a2_ref[...] = a2

    hot1 = (col == a1).astype(jnp.float32)
    hot2 = (col == a2).astype(jnp.float32)
    hit = hot1 + hot2
    rr = lax.broadcasted_iota(jnp.int32, (TB, TB), 0)
    cc = lax.broadcasted_iota(jnp.int32, (TB, TB), 1)
    tri = (rr > cc).astype(jnp.float32)
    carry = jnp.zeros((1, e), jnp.float32)
    pos_blocks = []
    for b in range(t // TB):
        hb = lax.slice(hit, (b * TB, 0), ((b + 1) * TB, e))
        posb = lax.dot_general(tri, hb, (((1,), (0,)), ((), ())),
                               preferred_element_type=jnp.float32) + carry
        carry = carry + jnp.sum(hb, axis=0, keepdims=True)
        pos_blocks.append(posb)
    pos = jnp.concatenate(pos_blocks, axis=0)
    cnt_ref[...] = jnp.minimum(carry, C)

    p1 = jnp.sum(pos * hot1, axis=1, keepdims=True).astype(jnp.int32)
    p2 = jnp.sum(pos * hot2, axis=1, keepdims=True).astype(jnp.int32)
    p1_ref[...] = p1
    p2_ref[...] = p2
    s1_ref[...] = a1 * C + jnp.minimum(p1, C - 1)
    s2_ref[...] = a2 * C + jnp.minimum(p2, C - 1)
    w1_ref[...] = jnp.where(p1 < C, w1, 0.0)
    w2_ref[...] = jnp.where(p2 < C, w2, 0.0)


def _route(x, gate_w):
    t, _ = x.shape
    e = gate_w.shape[0]
    f32 = jnp.float32
    i32 = jnp.int32
    return pl.pallas_call(
        _router_kernel,
        out_shape=(
            jax.ShapeDtypeStruct((t, e), f32),
            jax.ShapeDtypeStruct((t, 1), i32),
            jax.ShapeDtypeStruct((t, 1), i32),
            jax.ShapeDtypeStruct((t, 1), i32),
            jax.ShapeDtypeStruct((t, 1), i32),
            jax.ShapeDtypeStruct((t, 1), i32),
            jax.ShapeDtypeStruct((t, 1), i32),
            jax.ShapeDtypeStruct((t, 1), f32),
            jax.ShapeDtypeStruct((t, 1), f32),
            jax.ShapeDtypeStruct((1, e), f32),
        ),
    )(x, gate_w)


def _make_dispatch(t, h, e_total):
    mesh = plsc.VectorSubcoreMesh(core_axis_name="c", subcore_axis_name="s")
    epw = e_total // 32

    @functools.partial(
        pl.kernel,
        out_type=jax.ShapeDtypeStruct((e_total * NT, TILE, h), jnp.float32),
        mesh=mesh,
        scratch_types=[
            pltpu.VMEM((t,), jnp.int32),
            pltpu.VMEM((t,), jnp.int32),
            pltpu.VMEM((t,), jnp.int32),
            pltpu.VMEM((t,), jnp.int32),
            pltpu.VMEM((NT, TILE), jnp.int32),
            pltpu.VMEM((TILE, h), jnp.float32),
            pltpu.SemaphoreType.DMA,
        ],
        compiler_params=pltpu.CompilerParams(needs_layout_passes=False),
    )
    def dispatch(a1_hbm, a2_hbm, p1_hbm, p2_hbm, x_hbm, xg_hbm,
                 a1_v, a2_v, p1_v, p2_v, idxb, rows_v, sem):
        cid = lax.axis_index("c")
        sid = lax.axis_index("s")
        wid = sid * 2 + cid
        pltpu.sync_copy(a1_hbm, a1_v)
        pltpu.sync_copy(a2_hbm, a2_v)
        pltpu.sync_copy(p1_hbm, p1_v)
        pltpu.sync_copy(p2_hbm, p2_v)
        zi = jnp.zeros((16,), jnp.int32)
        for el in range(epw):
            e = wid * epw + el
            for r in range(NT):
                for q in range(TILE // 16):
                    idxb[r, pl.ds(q * 16, 16)] = zi
            e_vec = zi + e

            def chunk(i, cntv):
                tok = lax.iota(jnp.int32, 16) + i * 16
                a1c = a1_v[pl.ds(i * 16, 16)]
                a2c = a2_v[pl.ds(i * 16, 16)]
                p1c = p1_v[pl.ds(i * 16, 16)]
                p2c = p2_v[pl.ds(i * 16, 16)]
                h1 = a1c == e_vec
                h2 = a2c == e_vec
                m1 = h1 & (p1c < C)
                m2 = h2 & (p2c < C)
                q1 = jnp.minimum(p1c, C - 1)
                q2 = jnp.minimum(p2c, C - 1)
                plsc.store_scatter(
                    idxb,
                    [lax.shift_right_logical(q1, 6),
                     jnp.bitwise_and(q1, TILE - 1)], tok, mask=m1)
                plsc.store_scatter(
                    idxb,
                    [lax.shift_right_logical(q2, 6),
                     jnp.bitwise_and(q2, TILE - 1)], tok, mask=m2)
                return cntv + plsc.all_reduce_population_count(h1 | h2)

            cntv = lax.fori_loop(0, t // 16, chunk, jnp.zeros((16,), jnp.int32))
            ne = jnp.minimum(jnp.max(cntv), C)
            for ci in range(NT):
                @pl.when(ci * TILE < ne)
                def _gather():
                    pltpu.async_copy(x_hbm.at[idxb.at[ci]], rows_v, sem).wait()
                    pltpu.sync_copy(rows_v, xg_hbm.at[e * NT + ci])

    return dispatch


def _ffn_kernel(cnt_ref, xg_ref, gw_ref, uw_ref, dw_ref, dg_ref):
    j = pl.program_id(1)
    ne = cnt_ref[pl.program_id(0)]

    @pl.when(j * TILE < ne)
    def _tile():
        xe = xg_ref[0]
        g = lax.dot_general(xe, gw_ref[0], (((1,), (1,)), ((), ())),
                            preferred_element_type=jnp.float32)
        u = lax.dot_general(xe, uw_ref[0], (((1,), (1,)), ((), ())),
                            preferred_element_type=jnp.float32)
        hdn = (g * jax.nn.sigmoid(g)) * u
        dg_ref[0] = lax.dot_general(hdn, dw_ref[0], (((1,), (1,)), ((), ())),
                                    preferred_element_type=jnp.float32)


def _ffn(cnt, xg, gate_proj_w, up_proj_w, down_proj_w):
    e, i, h = gate_proj_w.shape

    def _active(ji, cnt_r, ei):
        nch = (cnt_r[ei] + TILE - 1) // TILE
        return jnp.minimum(ji, jnp.maximum(nch - 1, 0))

    grid_spec = pltpu.PrefetchScalarGridSpec(
        num_scalar_prefetch=1,
        grid=(e, NT),
        in_specs=[
            pl.BlockSpec((1, TILE, h),
                         lambda ei, ji, cnt_r: (ei * NT + _active(ji, cnt_r, ei), 0, 0)),
            pl.BlockSpec((1, i, h), lambda ei, ji, cnt_r: (ei, 0, 0)),
            pl.BlockSpec((1, i, h), lambda ei, ji, cnt_r: (ei, 0, 0)),
            pl.BlockSpec((1, h, i), lambda ei, ji, cnt_r: (ei, 0, 0)),
        ],
        out_specs=pl.BlockSpec(
            (1, TILE, h),
            lambda ei, ji, cnt_r: (ei * NT + _active(ji, cnt_r, ei), 0, 0)),
    )
    return pl.pallas_call(
        _ffn_kernel,
        grid_spec=grid_spec,
        out_shape=jax.ShapeDtypeStruct((e * NT, TILE, h), jnp.float32),
        compiler_params=pltpu.CompilerParams(
            dimension_semantics=("arbitrary", "arbitrary")),
    )(cnt, xg, gate_proj_w, up_proj_w, down_proj_w)


def _make_combine(t, h):
    mesh = plsc.VectorSubcoreMesh(core_axis_name="c", subcore_axis_name="s")
    tpw = t // 32
    ck = 32

    @functools.partial(
        pl.kernel,
        out_type=jax.ShapeDtypeStruct((t, h), jnp.float32),
        mesh=mesh,
        scratch_types=[
            pltpu.VMEM((tpw // ck, ck), jnp.int32),
            pltpu.VMEM((tpw // ck, ck), jnp.int32),
            pltpu.VMEM((tpw,), jnp.float32),
            pltpu.VMEM((tpw,), jnp.float32),
            pltpu.VMEM((ck, h), jnp.float32),
            pltpu.VMEM((ck, h), jnp.float32),
            pltpu.SemaphoreType.DMA,
        ],
        compiler_params=pltpu.CompilerParams(needs_layout_passes=False),
    )
    def combine(dg_hbm, s1_hbm, s2_hbm, w1_hbm, w2_hbm, out_hbm,
                s1b, s2b, w1b, w2b, rows1, rows2, sem):
        cid = lax.axis_index("c")
        sid = lax.axis_index("s")
        wid = sid * 2 + cid
        base = wid * tpw
        for k in range(tpw // ck):
            pltpu.sync_copy(s1_hbm.at[pl.ds(base + k * ck, ck)], s1b.at[k])
            pltpu.sync_copy(s2_hbm.at[pl.ds(base + k * ck, ck)], s2b.at[k])
        pltpu.sync_copy(w1_hbm.at[pl.ds(base, tpw)], w1b)
        pltpu.sync_copy(w2_hbm.at[pl.ds(base, tpw)], w2b)
        zf = jnp.zeros((16,), jnp.float32)
        for k in range(tpw // ck):
            pltpu.async_copy(dg_hbm.at[s1b.at[k]], rows1, sem).wait()
            pltpu.async_copy(dg_hbm.at[s2b.at[k]], rows2, sem).wait()

            zi = jnp.zeros((16,), jnp.int32)

            def row(r, carry):
                ridx = zi + (k * ck + r)
                av = plsc.load_gather(w1b, [ridx])
                bv = plsc.load_gather(w2b, [ridx])
                for q in range(h // 16):
                    sl = pl.ds(q * 16, 16)
                    rows1[r, sl] = av * rows1[r, sl] + bv * rows2[r, sl]
                return carry
            lax.fori_loop(0, ck, row, 0)
            pltpu.sync_copy(rows1, out_hbm.at[pl.ds(base + k * ck, ck), :])

    return combine


def kernel(hidden_states, gate_w, gate_proj_w, up_proj_w, down_proj_w):
    b, s, h = hidden_states.shape
    e = gate_w.shape[0]
    t = b * s
    x = hidden_states.reshape(t, h)
    (logits, a1, a2, p1, p2, s1, s2, w1, w2, cntf) = _route(x, gate_w)
    cnt = cntf[0].astype(jnp.int32)
    xg = _make_dispatch(t, h, e)(a1[:, 0], a2[:, 0], p1[:, 0], p2[:, 0], x)
    dg = _ffn(cnt, xg, gate_proj_w, up_proj_w, down_proj_w)
    final = _make_combine(t, h)(
        dg.reshape(e * C, h), s1[:, 0], s2[:, 0], w1[:, 0], w2[:, 0])
    return final.reshape(b, s, h), logits


# speedup vs baseline: 3.1498x; 1.0948x over previous
import functools

import jax
import jax.numpy as jnp
from jax import lax
from jax.experimental import pallas as pl
from jax.experimental.pallas import tpu as pltpu
from jax.experimental.pallas import tpu_sc as plsc

C = 256
TILE = 64
NT = C // TILE
TB = 256


def _router_kernel(x_ref, gw_ref, logits_ref, a1_ref, a2_ref, p1_ref, p2_ref,
                   s1_ref, s2_ref, w1_ref, w2_ref, cnt_ref):
    x = x_ref[...]
    gw = gw_ref[...]
    logits = lax.dot_general(x, gw, (((1,), (1,)), ((), ())),
                             preferred_element_type=jnp.float32)
    logits_ref[...] = logits
    e = gw.shape[0]
    t = logits.shape[0]
    col = lax.broadcasted_iota(jnp.int32, logits.shape, 1)
    m1 = jnp.max(logits, axis=1, keepdims=True)
    a1 = jnp.min(jnp.where(logits == m1, col, e), axis=1, keepdims=True)
    l2 = jnp.where(col == a1, -jnp.inf, logits)
    m2 = jnp.max(l2, axis=1, keepdims=True)
    a2 = jnp.min(jnp.where(l2 == m2, col, e), axis=1, keepdims=True)
    s2 = jnp.exp(m2 - m1)
    w1 = 1.0 / (1.0 + s2)
    w2 = s2 * w1
    a1_ref[...] = a1
    a2_ref[...] = a2

    hot1 = (col == a1).astype(jnp.float32)
    hot2 = (col == a2).astype(jnp.float32)
    hit = hot1 + hot2
    rr = lax.broadcasted_iota(jnp.int32, (TB, TB), 0)
    cc = lax.broadcasted_iota(jnp.int32, (TB, TB), 1)
    tri = (rr > cc).astype(jnp.float32)
    carry = jnp.zeros((1, e), jnp.float32)
    pos_blocks = []
    for b in range(t // TB):
        hb = lax.slice(hit, (b * TB, 0), ((b + 1) * TB, e))
        posb = lax.dot_general(tri, hb, (((1,), (0,)), ((), ())),
                               preferred_element_type=jnp.float32) + carry
        carry = carry + jnp.sum(hb, axis=0, keepdims=True)
        pos_blocks.append(posb)
    pos = jnp.concatenate(pos_blocks, axis=0)
    cnt_ref[...] = jnp.minimum(carry, C)

    p1 = jnp.sum(pos * hot1, axis=1, keepdims=True).astype(jnp.int32)
    p2 = jnp.sum(pos * hot2, axis=1, keepdims=True).astype(jnp.int32)
    p1_ref[...] = p1
    p2_ref[...] = p2
    s1_ref[...] = a1 * C + jnp.minimum(p1, C - 1)
    s2_ref[...] = a2 * C + jnp.minimum(p2, C - 1)
    w1_ref[...] = jnp.where(p1 < C, w1, 0.0)
    w2_ref[...] = jnp.where(p2 < C, w2, 0.0)


def _route(x, gate_w):
    t, _ = x.shape
    e = gate_w.shape[0]
    f32 = jnp.float32
    i32 = jnp.int32
    return pl.pallas_call(
        _router_kernel,
        out_shape=(
            jax.ShapeDtypeStruct((t, e), f32),
            jax.ShapeDtypeStruct((t, 1), i32),
            jax.ShapeDtypeStruct((t, 1), i32),
            jax.ShapeDtypeStruct((t, 1), i32),
            jax.ShapeDtypeStruct((t, 1), i32),
            jax.ShapeDtypeStruct((t, 1), i32),
            jax.ShapeDtypeStruct((t, 1), i32),
            jax.ShapeDtypeStruct((t, 1), f32),
            jax.ShapeDtypeStruct((t, 1), f32),
            jax.ShapeDtypeStruct((1, e), f32),
        ),
    )(x, gate_w)


def _make_dispatch(t, h, e_total):
    mesh = plsc.VectorSubcoreMesh(core_axis_name="c", subcore_axis_name="s")
    epw = e_total // 32
    GR = 32
    NS = C // GR

    @functools.partial(
        pl.kernel,
        out_type=jax.ShapeDtypeStruct((e_total * (C // GR), GR, h), jnp.float32),
        mesh=mesh,
        scratch_types=[
            pltpu.VMEM((t,), jnp.int32),
            pltpu.VMEM((t,), jnp.int32),
            pltpu.VMEM((t,), jnp.int32),
            pltpu.VMEM((t,), jnp.int32),
            pltpu.VMEM((epw * NS, GR), jnp.int32),
            pltpu.VMEM((GR, h), jnp.float32),
            pltpu.VMEM((GR, h), jnp.float32),
            pltpu.SemaphoreType.DMA,
            pltpu.SemaphoreType.DMA,
            pltpu.SemaphoreType.DMA,
            pltpu.SemaphoreType.DMA,
            pltpu.SemaphoreType.DMA,
        ],
        compiler_params=pltpu.CompilerParams(needs_layout_passes=False),
    )
    def dispatch(a1_hbm, a2_hbm, p1_hbm, p2_hbm, x_hbm, xg_hbm,
                 a1_v, a2_v, p1_v, p2_v, idxb, rows0, rows1,
                 isem, gsem0, gsem1, wsem0, wsem1):
        cid = lax.axis_index("c")
        sid = lax.axis_index("s")
        wid = sid * 2 + cid
        cps = [pltpu.async_copy(a1_hbm, a1_v, isem),
               pltpu.async_copy(a2_hbm, a2_v, isem),
               pltpu.async_copy(p1_hbm, p1_v, isem),
               pltpu.async_copy(p2_hbm, p2_v, isem)]
        for cp in cps:
            cp.wait()
        zi = jnp.zeros((16,), jnp.int32)
        nes = []
        for el in range(epw):
            e = wid * epw + el
            for r in range(NS):
                for q in range(GR // 16):
                    idxb[el * NS + r, pl.ds(q * 16, 16)] = zi
            e_vec = zi + e

            def chunk(i, cntv):
                tok = lax.iota(jnp.int32, 16) + i * 16
                a1c = a1_v[pl.ds(i * 16, 16)]
                a2c = a2_v[pl.ds(i * 16, 16)]
                p1c = p1_v[pl.ds(i * 16, 16)]
                p2c = p2_v[pl.ds(i * 16, 16)]
                h1 = a1c == e_vec
                h2 = a2c == e_vec
                m1 = h1 & (p1c < C)
                m2 = h2 & (p2c < C)
                q1 = jnp.minimum(p1c, C - 1)
                q2 = jnp.minimum(p2c, C - 1)
                base = zi + el * NS
                plsc.store_scatter(
                    idxb,
                    [base + lax.shift_right_logical(q1, 5),
                     jnp.bitwise_and(q1, GR - 1)], tok, mask=m1)
                plsc.store_scatter(
                    idxb,
                    [base + lax.shift_right_logical(q2, 5),
                     jnp.bitwise_and(q2, GR - 1)], tok, mask=m2)
                return cntv + plsc.all_reduce_population_count(h1 | h2)

            cntv = lax.fori_loop(0, t // 16, chunk, jnp.zeros((16,), jnp.int32))
            nes.append(jnp.minimum(jnp.max(cntv), C))

        rows = (rows0, rows1)
        gsem = (gsem0, gsem1)
        wsem = (wsem0, wsem1)
        for el in range(epw):
            e = wid * epw + el
            ne = nes[el]
            acts = [sc * GR < ne for sc in range(NS)]
            gds = [pltpu.make_async_copy(
                x_hbm.at[idxb.at[el * NS + sc]], rows[sc % 2], gsem[sc % 2])
                for sc in range(NS)]
            wbs = [pltpu.make_async_copy(
                rows[sc % 2], xg_hbm.at[e * NS + sc], wsem[sc % 2])
                for sc in range(NS)]
            for sc in range(NS):
                @pl.when(acts[sc])
                def _slot(sc=sc):
                    gds[sc].start()
                    gds[sc].wait()
                    if sc > 0:
                        wbs[sc - 1].wait()
                    wbs[sc].start()
            for sc in range(NS):
                last = acts[sc] if sc == NS - 1 else (acts[sc] & ~acts[sc + 1])

                @pl.when(last)
                def _drain(sc=sc):
                    wbs[sc].wait()

    return dispatch


def _ffn_kernel(cnt_ref, xg_ref, gw_ref, uw_ref, dw_ref, dg_ref):
    j = pl.program_id(1)
    ne = cnt_ref[pl.program_id(0)]

    @pl.when(j * TILE < ne)
    def _tile():
        xe = xg_ref[0]
        g = lax.dot_general(xe, gw_ref[0], (((1,), (1,)), ((), ())),
                            preferred_element_type=jnp.float32)
        u = lax.dot_general(xe, uw_ref[0], (((1,), (1,)), ((), ())),
                            preferred_element_type=jnp.float32)
        hdn = (g * jax.nn.sigmoid(g)) * u
        dg_ref[0] = lax.dot_general(hdn, dw_ref[0], (((1,), (1,)), ((), ())),
                                    preferred_element_type=jnp.float32)


def _ffn(cnt, xg, gate_proj_w, up_proj_w, down_proj_w):
    e, i, h = gate_proj_w.shape

    def _active(ji, cnt_r, ei):
        nch = (cnt_r[ei] + TILE - 1) // TILE
        return jnp.minimum(ji, jnp.maximum(nch - 1, 0))

    grid_spec = pltpu.PrefetchScalarGridSpec(
        num_scalar_prefetch=1,
        grid=(e, NT),
        in_specs=[
            pl.BlockSpec((1, TILE, h),
                         lambda ei, ji, cnt_r: (ei * NT + _active(ji, cnt_r, ei), 0, 0)),
            pl.BlockSpec((1, i, h), lambda ei, ji, cnt_r: (ei, 0, 0)),
            pl.BlockSpec((1, i, h), lambda ei, ji, cnt_r: (ei, 0, 0)),
            pl.BlockSpec((1, h, i), lambda ei, ji, cnt_r: (ei, 0, 0)),
        ],
        out_specs=pl.BlockSpec(
            (1, TILE, h),
            lambda ei, ji, cnt_r: (ei * NT + _active(ji, cnt_r, ei), 0, 0)),
    )
    return pl.pallas_call(
        _ffn_kernel,
        grid_spec=grid_spec,
        out_shape=jax.ShapeDtypeStruct((e * NT, TILE, h), jnp.float32),
        compiler_params=pltpu.CompilerParams(
            dimension_semantics=("arbitrary", "arbitrary")),
    )(cnt, xg, gate_proj_w, up_proj_w, down_proj_w)


def _make_combine(t, h):
    mesh = plsc.VectorSubcoreMesh(core_axis_name="c", subcore_axis_name="s")
    tpw = t // 32
    ck = 32

    @functools.partial(
        pl.kernel,
        out_type=jax.ShapeDtypeStruct((t, h), jnp.float32),
        mesh=mesh,
        scratch_types=[
            pltpu.VMEM((tpw // ck, ck), jnp.int32),
            pltpu.VMEM((tpw // ck, ck), jnp.int32),
            pltpu.VMEM((tpw,), jnp.float32),
            pltpu.VMEM((tpw,), jnp.float32),
            pltpu.VMEM((ck, h), jnp.float32),
            pltpu.VMEM((ck, h), jnp.float32),
            pltpu.SemaphoreType.DMA,
        ],
        compiler_params=pltpu.CompilerParams(needs_layout_passes=False),
    )
    def combine(dg_hbm, s1_hbm, s2_hbm, w1_hbm, w2_hbm, out_hbm,
                s1b, s2b, w1b, w2b, rows1, rows2, sem):
        cid = lax.axis_index("c")
        sid = lax.axis_index("s")
        wid = sid * 2 + cid
        base = wid * tpw
        for k in range(tpw // ck):
            pltpu.sync_copy(s1_hbm.at[pl.ds(base + k * ck, ck)], s1b.at[k])
            pltpu.sync_copy(s2_hbm.at[pl.ds(base + k * ck, ck)], s2b.at[k])
        pltpu.sync_copy(w1_hbm.at[pl.ds(base, tpw)], w1b)
        pltpu.sync_copy(w2_hbm.at[pl.ds(base, tpw)], w2b)
        zf = jnp.zeros((16,), jnp.float32)
        for k in range(tpw // ck):
            pltpu.async_copy(dg_hbm.at[s1b.at[k]], rows1, sem).wait()
            pltpu.async_copy(dg_hbm.at[s2b.at[k]], rows2, sem).wait()

            zi = jnp.zeros((16,), jnp.int32)

            def row(r, carry):
                ridx = zi + (k * ck + r)
                av = plsc.load_gather(w1b, [ridx])
                bv = plsc.load_gather(w2b, [ridx])
                for q in range(h // 16):
                    sl = pl.ds(q * 16, 16)
                    rows1[r, sl] = av * rows1[r, sl] + bv * rows2[r, sl]
                return carry
            lax.fori_loop(0, ck, row, 0)
            pltpu.sync_copy(rows1, out_hbm.at[pl.ds(base + k * ck, ck), :])

    return combine


def kernel(hidden_states, gate_w, gate_proj_w, up_proj_w, down_proj_w):
    b, s, h = hidden_states.shape
    e = gate_w.shape[0]
    t = b * s
    x = hidden_states.reshape(t, h)
    (logits, a1, a2, p1, p2, s1, s2, w1, w2, cntf) = _route(x, gate_w)
    cnt = cntf[0].astype(jnp.int32)
    xg = _make_dispatch(t, h, e)(a1[:, 0], a2[:, 0], p1[:, 0], p2[:, 0], x)
    dg = _ffn(cnt, xg.reshape(e * NT, TILE, h),
              gate_proj_w, up_proj_w, down_proj_w)
    final = _make_combine(t, h)(
        dg.reshape(e * C, h), s1[:, 0], s2[:, 0], w1[:, 0], w2[:, 0])
    return final.reshape(b, s, h), logits


# speedup vs baseline: 3.4175x; 1.0850x over previous
import functools

import jax
import jax.numpy as jnp
from jax import lax
from jax.experimental import pallas as pl
from jax.experimental.pallas import tpu as pltpu
from jax.experimental.pallas import tpu_sc as plsc

C = 256
TILE = 128
NT = C // TILE
TB = 256


def _router_kernel(x_ref, gw_ref, logits_ref, a1_ref, a2_ref, p1_ref, p2_ref,
                   s1_ref, s2_ref, w1_ref, w2_ref, cnt_ref):
    x = x_ref[...]
    gw = gw_ref[...]
    logits = lax.dot_general(x, gw, (((1,), (1,)), ((), ())),
                             preferred_element_type=jnp.float32)
    logits_ref[...] = logits
    e = gw.shape[0]
    t = logits.shape[0]
    col = lax.broadcasted_iota(jnp.int32, logits.shape, 1)
    m1 = jnp.max(logits, axis=1, keepdims=True)
    a1 = jnp.min(jnp.where(logits == m1, col, e), axis=1, keepdims=True)
    l2 = jnp.where(col == a1, -jnp.inf, logits)
    m2 = jnp.max(l2, axis=1, keepdims=True)
    a2 = jnp.min(jnp.where(l2 == m2, col, e), axis=1, keepdims=True)
    s2 = jnp.exp(m2 - m1)
    w1 = 1.0 / (1.0 + s2)
    w2 = s2 * w1
    a1_ref[...] = a1
    a2_ref[...] = a2

    hot1 = (col == a1).astype(jnp.float32)
    hot2 = (col == a2).astype(jnp.float32)
    hit = hot1 + hot2
    rr = lax.broadcasted_iota(jnp.int32, (TB, TB), 0)
    cc = lax.broadcasted_iota(jnp.int32, (TB, TB), 1)
    tri = (rr > cc).astype(jnp.float32)
    carry = jnp.zeros((1, e), jnp.float32)
    pos_blocks = []
    for b in range(t // TB):
        hb = lax.slice(hit, (b * TB, 0), ((b + 1) * TB, e))
        posb = lax.dot_general(tri, hb, (((1,), (0,)), ((), ())),
                               preferred_element_type=jnp.float32) + carry
        carry = carry + jnp.sum(hb, axis=0, keepdims=True)
        pos_blocks.append(posb)
    pos = jnp.concatenate(pos_blocks, axis=0)
    cnt_ref[...] = jnp.minimum(carry, C)

    p1 = jnp.sum(pos * hot1, axis=1, keepdims=True).astype(jnp.int32)
    p2 = jnp.sum(pos * hot2, axis=1, keepdims=True).astype(jnp.int32)
    p1_ref[...] = p1
    p2_ref[...] = p2
    s1_ref[...] = a1 * C + jnp.minimum(p1, C - 1)
    s2_ref[...] = a2 * C + jnp.minimum(p2, C - 1)
    w1_ref[...] = jnp.where(p1 < C, w1, 0.0)
    w2_ref[...] = jnp.where(p2 < C, w2, 0.0)


def _route(x, gate_w):
    t, _ = x.shape
    e = gate_w.shape[0]
    f32 = jnp.float32
    i32 = jnp.int32
    return pl.pallas_call(
        _router_kernel,
        out_shape=(
            jax.ShapeDtypeStruct((t, e), f32),
            jax.ShapeDtypeStruct((t, 1), i32),
            jax.ShapeDtypeStruct((t, 1), i32),
            jax.ShapeDtypeStruct((t, 1), i32),
            jax.ShapeDtypeStruct((t, 1), i32),
            jax.ShapeDtypeStruct((t, 1), i32),
            jax.ShapeDtypeStruct((t, 1), i32),
            jax.ShapeDtypeStruct((t, 1), f32),
            jax.ShapeDtypeStruct((t, 1), f32),
            jax.ShapeDtypeStruct((1, e), f32),
        ),
    )(x, gate_w)


def _make_dispatch(t, h, e_total):
    mesh = plsc.VectorSubcoreMesh(core_axis_name="c", subcore_axis_name="s")
    epw = e_total // 32
    GR = 32
    NS = C // GR

    @functools.partial(
        pl.kernel,
        out_type=jax.ShapeDtypeStruct((e_total * (C // GR), GR, h), jnp.float32),
        mesh=mesh,
        scratch_types=[
            pltpu.VMEM((t,), jnp.int32),
            pltpu.VMEM((t,), jnp.int32),
            pltpu.VMEM((t,), jnp.int32),
            pltpu.VMEM((t,), jnp.int32),
            pltpu.VMEM((epw * NS, GR), jnp.int32),
            pltpu.VMEM((GR, h), jnp.float32),
            pltpu.VMEM((GR, h), jnp.float32),
            pltpu.SemaphoreType.DMA,
            pltpu.SemaphoreType.DMA,
            pltpu.SemaphoreType.DMA,
            pltpu.SemaphoreType.DMA,
            pltpu.SemaphoreType.DMA,
        ],
        compiler_params=pltpu.CompilerParams(needs_layout_passes=False),
    )
    def dispatch(a1_hbm, a2_hbm, p1_hbm, p2_hbm, x_hbm, xg_hbm,
                 a1_v, a2_v, p1_v, p2_v, idxb, rows0, rows1,
                 isem, gsem0, gsem1, wsem0, wsem1):
        cid = lax.axis_index("c")
        sid = lax.axis_index("s")
        wid = sid * 2 + cid
        cps = [pltpu.async_copy(a1_hbm, a1_v, isem),
               pltpu.async_copy(a2_hbm, a2_v, isem),
               pltpu.async_copy(p1_hbm, p1_v, isem),
               pltpu.async_copy(p2_hbm, p2_v, isem)]
        for cp in cps:
            cp.wait()
        zi = jnp.zeros((16,), jnp.int32)
        nes = []
        for el in range(epw):
            e = wid * epw + el
            for r in range(NS):
                for q in range(GR // 16):
                    idxb[el * NS + r, pl.ds(q * 16, 16)] = zi
            e_vec = zi + e

            def chunk(i, cntv):
                tok = lax.iota(jnp.int32, 16) + i * 16
                a1c = a1_v[pl.ds(i * 16, 16)]
                a2c = a2_v[pl.ds(i * 16, 16)]
                p1c = p1_v[pl.ds(i * 16, 16)]
                p2c = p2_v[pl.ds(i * 16, 16)]
                h1 = a1c == e_vec
                h2 = a2c == e_vec
                m1 = h1 & (p1c < C)
                m2 = h2 & (p2c < C)
                q1 = jnp.minimum(p1c, C - 1)
                q2 = jnp.minimum(p2c, C - 1)
                base = zi + el * NS
                plsc.store_scatter(
                    idxb,
                    [base + lax.shift_right_logical(q1, 5),
                     jnp.bitwise_and(q1, GR - 1)], tok, mask=m1)
                plsc.store_scatter(
                    idxb,
                    [base + lax.shift_right_logical(q2, 5),
                     jnp.bitwise_and(q2, GR - 1)], tok, mask=m2)
                return cntv + plsc.all_reduce_population_count(h1 | h2)

            cntv = lax.fori_loop(0, t // 16, chunk, jnp.zeros((16,), jnp.int32))
            nes.append(jnp.minimum(jnp.max(cntv), C))

        rows = (rows0, rows1)
        gsem = (gsem0, gsem1)
        wsem = (wsem0, wsem1)
        for el in range(epw):
            e = wid * epw + el
            ne = nes[el]
            acts = [sc * GR < ne for sc in range(NS)]
            gds = [pltpu.make_async_copy(
                x_hbm.at[idxb.at[el * NS + sc]], rows[sc % 2], gsem[sc % 2])
                for sc in range(NS)]
            wbs = [pltpu.make_async_copy(
                rows[sc % 2], xg_hbm.at[e * NS + sc], wsem[sc % 2])
                for sc in range(NS)]
            for sc in range(NS):
                @pl.when(acts[sc])
                def _slot(sc=sc):
                    gds[sc].start()
                    gds[sc].wait()
                    if sc > 0:
                        wbs[sc - 1].wait()
                    wbs[sc].start()
            for sc in range(NS):
                last = acts[sc] if sc == NS - 1 else (acts[sc] & ~acts[sc + 1])

                @pl.when(last)
                def _drain(sc=sc):
                    wbs[sc].wait()

    return dispatch


def _ffn_kernel(cnt_ref, xg_ref, gw_ref, uw_ref, dw_ref, dg_ref):
    j = pl.program_id(1)
    ne = cnt_ref[pl.program_id(0)]

    @pl.when(j * TILE < ne)
    def _tile():
        xe = xg_ref[0]
        g = lax.dot_general(xe, gw_ref[0], (((1,), (1,)), ((), ())),
                            preferred_element_type=jnp.float32)
        u = lax.dot_general(xe, uw_ref[0], (((1,), (1,)), ((), ())),
                            preferred_element_type=jnp.float32)
        hdn = (g * jax.nn.sigmoid(g)) * u
        dg_ref[0] = lax.dot_general(hdn, dw_ref[0], (((1,), (1,)), ((), ())),
                                    preferred_element_type=jnp.float32)


def _ffn(cnt, xg, gate_proj_w, up_proj_w, down_proj_w):
    e, i, h = gate_proj_w.shape

    def _active(ji, cnt_r, ei):
        nch = (cnt_r[ei] + TILE - 1) // TILE
        return jnp.minimum(ji, jnp.maximum(nch - 1, 0))

    grid_spec = pltpu.PrefetchScalarGridSpec(
        num_scalar_prefetch=1,
        grid=(e, NT),
        in_specs=[
            pl.BlockSpec((1, TILE, h),
                         lambda ei, ji, cnt_r: (ei * NT + _active(ji, cnt_r, ei), 0, 0)),
            pl.BlockSpec((1, i, h), lambda ei, ji, cnt_r: (ei, 0, 0)),
            pl.BlockSpec((1, i, h), lambda ei, ji, cnt_r: (ei, 0, 0)),
            pl.BlockSpec((1, h, i), lambda ei, ji, cnt_r: (ei, 0, 0)),
        ],
        out_specs=pl.BlockSpec(
            (1, TILE, h),
            lambda ei, ji, cnt_r: (ei * NT + _active(ji, cnt_r, ei), 0, 0)),
    )
    return pl.pallas_call(
        _ffn_kernel,
        grid_spec=grid_spec,
        out_shape=jax.ShapeDtypeStruct((e * NT, TILE, h), jnp.float32),
        compiler_params=pltpu.CompilerParams(
            dimension_semantics=("arbitrary", "arbitrary")),
    )(cnt, xg, gate_proj_w, up_proj_w, down_proj_w)


def _make_combine(t, h):
    mesh = plsc.VectorSubcoreMesh(core_axis_name="c", subcore_axis_name="s")
    tpw = t // 32
    ck = 32

    @functools.partial(
        pl.kernel,
        out_type=jax.ShapeDtypeStruct((t, h), jnp.float32),
        mesh=mesh,
        scratch_types=[
            pltpu.VMEM((tpw // ck, ck), jnp.int32),
            pltpu.VMEM((tpw // ck, ck), jnp.int32),
            pltpu.VMEM((tpw,), jnp.float32),
            pltpu.VMEM((tpw,), jnp.float32),
            pltpu.VMEM((ck, h), jnp.float32),
            pltpu.VMEM((ck, h), jnp.float32),
            pltpu.SemaphoreType.DMA,
        ],
        compiler_params=pltpu.CompilerParams(needs_layout_passes=False),
    )
    def combine(dg_hbm, s1_hbm, s2_hbm, w1_hbm, w2_hbm, out_hbm,
                s1b, s2b, w1b, w2b, rows1, rows2, sem):
        cid = lax.axis_index("c")
        sid = lax.axis_index("s")
        wid = sid * 2 + cid
        base = wid * tpw
        for k in range(tpw // ck):
            pltpu.sync_copy(s1_hbm.at[pl.ds(base + k * ck, ck)], s1b.at[k])
            pltpu.sync_copy(s2_hbm.at[pl.ds(base + k * ck, ck)], s2b.at[k])
        pltpu.sync_copy(w1_hbm.at[pl.ds(base, tpw)], w1b)
        pltpu.sync_copy(w2_hbm.at[pl.ds(base, tpw)], w2b)
        zf = jnp.zeros((16,), jnp.float32)
        for k in range(tpw // ck):
            pltpu.async_copy(dg_hbm.at[s1b.at[k]], rows1, sem).wait()
            pltpu.async_copy(dg_hbm.at[s2b.at[k]], rows2, sem).wait()

            zi = jnp.zeros((16,), jnp.int32)

            def row(r, carry):
                ridx = zi + (k * ck + r)
                av = plsc.load_gather(w1b, [ridx])
                bv = plsc.load_gather(w2b, [ridx])
                for q in range(h // 16):
                    sl = pl.ds(q * 16, 16)
                    rows1[r, sl] = av * rows1[r, sl] + bv * rows2[r, sl]
                return carry
            lax.fori_loop(0, ck, row, 0)
            pltpu.sync_copy(rows1, out_hbm.at[pl.ds(base + k * ck, ck), :])

    return combine


def kernel(hidden_states, gate_w, gate_proj_w, up_proj_w, down_proj_w):
    b, s, h = hidden_states.shape
    e = gate_w.shape[0]
    t = b * s
    x = hidden_states.reshape(t, h)
    (logits, a1, a2, p1, p2, s1, s2, w1, w2, cntf) = _route(x, gate_w)
    cnt = cntf[0].astype(jnp.int32)
    xg = _make_dispatch(t, h, e)(a1[:, 0], a2[:, 0], p1[:, 0], p2[:, 0], x)
    dg = _ffn(cnt, xg.reshape(e * NT, TILE, h),
              gate_proj_w, up_proj_w, down_proj_w)
    final = _make_combine(t, h)(
        dg.reshape(e * C, h), s1[:, 0], s2[:, 0], w1[:, 0], w2[:, 0])
    return final.reshape(b, s, h), logits


# speedup vs baseline: 4.9519x; 1.4490x over previous
import functools

import jax
import jax.numpy as jnp
from jax import lax
from jax.experimental import pallas as pl
from jax.experimental.pallas import tpu as pltpu
from jax.experimental.pallas import tpu_sc as plsc

C = 256
TILE = 256
NT = C // TILE
TB = 256


def _router_kernel(x_ref, gw_ref, logits_ref, a1_ref, a2_ref, p1_ref, p2_ref,
                   s1_ref, s2_ref, w1_ref, w2_ref, cnt_ref):
    x = x_ref[...]
    gw = gw_ref[...]
    logits = lax.dot_general(x, gw, (((1,), (1,)), ((), ())),
                             preferred_element_type=jnp.float32)
    logits_ref[...] = logits
    e = gw.shape[0]
    t = logits.shape[0]
    col = lax.broadcasted_iota(jnp.int32, logits.shape, 1)
    m1 = jnp.max(logits, axis=1, keepdims=True)
    a1 = jnp.min(jnp.where(logits == m1, col, e), axis=1, keepdims=True)
    l2 = jnp.where(col == a1, -jnp.inf, logits)
    m2 = jnp.max(l2, axis=1, keepdims=True)
    a2 = jnp.min(jnp.where(l2 == m2, col, e), axis=1, keepdims=True)
    s2 = jnp.exp(m2 - m1)
    w1 = 1.0 / (1.0 + s2)
    w2 = s2 * w1
    a1_ref[...] = a1
    a2_ref[...] = a2

    hot1 = (col == a1).astype(jnp.float32)
    hot2 = (col == a2).astype(jnp.float32)
    hit = hot1 + hot2
    rr = lax.broadcasted_iota(jnp.int32, (TB, TB), 0)
    cc = lax.broadcasted_iota(jnp.int32, (TB, TB), 1)
    tri = (rr > cc).astype(jnp.float32)
    carry = jnp.zeros((1, e), jnp.float32)
    pos_blocks = []
    for b in range(t // TB):
        hb = lax.slice(hit, (b * TB, 0), ((b + 1) * TB, e))
        posb = lax.dot_general(tri, hb, (((1,), (0,)), ((), ())),
                               preferred_element_type=jnp.float32) + carry
        carry = carry + jnp.sum(hb, axis=0, keepdims=True)
        pos_blocks.append(posb)
    pos = jnp.concatenate(pos_blocks, axis=0)
    cnt_ref[...] = jnp.minimum(carry, C)

    p1 = jnp.sum(pos * hot1, axis=1, keepdims=True).astype(jnp.int32)
    p2 = jnp.sum(pos * hot2, axis=1, keepdims=True).astype(jnp.int32)
    p1_ref[...] = p1
    p2_ref[...] = p2
    s1_ref[...] = a1 * C + jnp.minimum(p1, C - 1)
    s2_ref[...] = a2 * C + jnp.minimum(p2, C - 1)
    w1_ref[...] = jnp.where(p1 < C, w1, 0.0)
    w2_ref[...] = jnp.where(p2 < C, w2, 0.0)


def _route(x, gate_w):
    t, _ = x.shape
    e = gate_w.shape[0]
    f32 = jnp.float32
    i32 = jnp.int32
    return pl.pallas_call(
        _router_kernel,
        out_shape=(
            jax.ShapeDtypeStruct((t, e), f32),
            jax.ShapeDtypeStruct((t, 1), i32),
            jax.ShapeDtypeStruct((t, 1), i32),
            jax.ShapeDtypeStruct((t, 1), i32),
            jax.ShapeDtypeStruct((t, 1), i32),
            jax.ShapeDtypeStruct((t, 1), i32),
            jax.ShapeDtypeStruct((t, 1), i32),
            jax.ShapeDtypeStruct((t, 1), f32),
            jax.ShapeDtypeStruct((t, 1), f32),
            jax.ShapeDtypeStruct((1, e), f32),
        ),
    )(x, gate_w)


def _make_dispatch(t, h, e_total):
    mesh = plsc.VectorSubcoreMesh(core_axis_name="c", subcore_axis_name="s")
    epw = e_total // 32
    GR = 32
    NS = C // GR

    @functools.partial(
        pl.kernel,
        out_type=jax.ShapeDtypeStruct((e_total * (C // GR), GR, h), jnp.float32),
        mesh=mesh,
        scratch_types=[
            pltpu.VMEM((t,), jnp.int32),
            pltpu.VMEM((t,), jnp.int32),
            pltpu.VMEM((t,), jnp.int32),
            pltpu.VMEM((t,), jnp.int32),
            pltpu.VMEM((epw * NS, GR), jnp.int32),
            pltpu.VMEM((GR, h), jnp.float32),
            pltpu.VMEM((GR, h), jnp.float32),
            pltpu.SemaphoreType.DMA,
            pltpu.SemaphoreType.DMA,
            pltpu.SemaphoreType.DMA,
            pltpu.SemaphoreType.DMA,
            pltpu.SemaphoreType.DMA,
        ],
        compiler_params=pltpu.CompilerParams(needs_layout_passes=False),
    )
    def dispatch(a1_hbm, a2_hbm, p1_hbm, p2_hbm, x_hbm, xg_hbm,
                 a1_v, a2_v, p1_v, p2_v, idxb, rows0, rows1,
                 isem, gsem0, gsem1, wsem0, wsem1):
        cid = lax.axis_index("c")
        sid = lax.axis_index("s")
        wid = sid * 2 + cid
        cps = [pltpu.async_copy(a1_hbm, a1_v, isem),
               pltpu.async_copy(a2_hbm, a2_v, isem),
               pltpu.async_copy(p1_hbm, p1_v, isem),
               pltpu.async_copy(p2_hbm, p2_v, isem)]
        for cp in cps:
            cp.wait()
        zi = jnp.zeros((16,), jnp.int32)
        nes = []
        for el in range(epw):
            e = wid * epw + el
            for r in range(NS):
                for q in range(GR // 16):
                    idxb[el * NS + r, pl.ds(q * 16, 16)] = zi
            e_vec = zi + e

            def chunk(i, cntv):
                tok = lax.iota(jnp.int32, 16) + i * 16
                a1c = a1_v[pl.ds(i * 16, 16)]
                a2c = a2_v[pl.ds(i * 16, 16)]
                p1c = p1_v[pl.ds(i * 16, 16)]
                p2c = p2_v[pl.ds(i * 16, 16)]
                h1 = a1c == e_vec
                h2 = a2c == e_vec
                m1 = h1 & (p1c < C)
                m2 = h2 & (p2c < C)
                q1 = jnp.minimum(p1c, C - 1)
                q2 = jnp.minimum(p2c, C - 1)
                base = zi + el * NS
                plsc.store_scatter(
                    idxb,
                    [base + lax.shift_right_logical(q1, 5),
                     jnp.bitwise_and(q1, GR - 1)], tok, mask=m1)
                plsc.store_scatter(
                    idxb,
                    [base + lax.shift_right_logical(q2, 5),
                     jnp.bitwise_and(q2, GR - 1)], tok, mask=m2)
                return cntv + plsc.all_reduce_population_count(h1 | h2)

            cntv = lax.fori_loop(0, t // 16, chunk, jnp.zeros((16,), jnp.int32))
            nes.append(jnp.minimum(jnp.max(cntv), C))

        rows = (rows0, rows1)
        gsem = (gsem0, gsem1)
        wsem = (wsem0, wsem1)
        for el in range(epw):
            e = wid * epw + el
            ne = nes[el]
            acts = [sc * GR < ne for sc in range(NS)]
            gds = [pltpu.make_async_copy(
                x_hbm.at[idxb.at[el * NS + sc]], rows[sc % 2], gsem[sc % 2])
                for sc in range(NS)]
            wbs = [pltpu.make_async_copy(
                rows[sc % 2], xg_hbm.at[e * NS + sc], wsem[sc % 2])
                for sc in range(NS)]
            for sc in range(NS):
                @pl.when(acts[sc])
                def _slot(sc=sc):
                    gds[sc].start()
                    gds[sc].wait()
                    if sc > 0:
                        wbs[sc - 1].wait()
                    wbs[sc].start()
            for sc in range(NS):
                last = acts[sc] if sc == NS - 1 else (acts[sc] & ~acts[sc + 1])

                @pl.when(last)
                def _drain(sc=sc):
                    wbs[sc].wait()

    return dispatch


def _ffn_kernel(cnt_ref, xg_ref, gw_ref, uw_ref, dw_ref, dg_ref):
    j = pl.program_id(1)
    ne = cnt_ref[pl.program_id(0)]

    @pl.when(j * TILE < ne)
    def _tile():
        xe = xg_ref[0]
        g = lax.dot_general(xe, gw_ref[0], (((1,), (1,)), ((), ())),
                            preferred_element_type=jnp.float32)
        u = lax.dot_general(xe, uw_ref[0], (((1,), (1,)), ((), ())),
                            preferred_element_type=jnp.float32)
        hdn = (g * jax.nn.sigmoid(g)) * u
        dg_ref[0] = lax.dot_general(hdn, dw_ref[0], (((1,), (1,)), ((), ())),
                                    preferred_element_type=jnp.float32)


def _ffn(cnt, xg, gate_proj_w, up_proj_w, down_proj_w):
    e, i, h = gate_proj_w.shape

    def _active(ji, cnt_r, ei):
        nch = (cnt_r[ei] + TILE - 1) // TILE
        return jnp.minimum(ji, jnp.maximum(nch - 1, 0))

    grid_spec = pltpu.PrefetchScalarGridSpec(
        num_scalar_prefetch=1,
        grid=(e, NT),
        in_specs=[
            pl.BlockSpec((1, TILE, h),
                         lambda ei, ji, cnt_r: (ei * NT + _active(ji, cnt_r, ei), 0, 0)),
            pl.BlockSpec((1, i, h), lambda ei, ji, cnt_r: (ei, 0, 0)),
            pl.BlockSpec((1, i, h), lambda ei, ji, cnt_r: (ei, 0, 0)),
            pl.BlockSpec((1, h, i), lambda ei, ji, cnt_r: (ei, 0, 0)),
        ],
        out_specs=pl.BlockSpec(
            (1, TILE, h),
            lambda ei, ji, cnt_r: (ei * NT + _active(ji, cnt_r, ei), 0, 0)),
    )
    return pl.pallas_call(
        _ffn_kernel,
        grid_spec=grid_spec,
        out_shape=jax.ShapeDtypeStruct((e * NT, TILE, h), jnp.float32),
        compiler_params=pltpu.CompilerParams(
            dimension_semantics=("arbitrary", "arbitrary")),
    )(cnt, xg, gate_proj_w, up_proj_w, down_proj_w)


def _make_combine(t, h):
    mesh = plsc.VectorSubcoreMesh(core_axis_name="c", subcore_axis_name="s")
    tpw = t // 32
    ck = 32

    @functools.partial(
        pl.kernel,
        out_type=jax.ShapeDtypeStruct((t, h), jnp.float32),
        mesh=mesh,
        scratch_types=[
            pltpu.VMEM((tpw // ck, ck), jnp.int32),
            pltpu.VMEM((tpw // ck, ck), jnp.int32),
            pltpu.VMEM((tpw,), jnp.float32),
            pltpu.VMEM((tpw,), jnp.float32),
            pltpu.VMEM((ck, h), jnp.float32),
            pltpu.VMEM((ck, h), jnp.float32),
            pltpu.SemaphoreType.DMA,
        ],
        compiler_params=pltpu.CompilerParams(needs_layout_passes=False),
    )
    def combine(dg_hbm, s1_hbm, s2_hbm, w1_hbm, w2_hbm, out_hbm,
                s1b, s2b, w1b, w2b, rows1, rows2, sem):
        cid = lax.axis_index("c")
        sid = lax.axis_index("s")
        wid = sid * 2 + cid
        base = wid * tpw
        for k in range(tpw // ck):
            pltpu.sync_copy(s1_hbm.at[pl.ds(base + k * ck, ck)], s1b.at[k])
            pltpu.sync_copy(s2_hbm.at[pl.ds(base + k * ck, ck)], s2b.at[k])
        pltpu.sync_copy(w1_hbm.at[pl.ds(base, tpw)], w1b)
        pltpu.sync_copy(w2_hbm.at[pl.ds(base, tpw)], w2b)
        zf = jnp.zeros((16,), jnp.float32)
        for k in range(tpw // ck):
            pltpu.async_copy(dg_hbm.at[s1b.at[k]], rows1, sem).wait()
            pltpu.async_copy(dg_hbm.at[s2b.at[k]], rows2, sem).wait()

            zi = jnp.zeros((16,), jnp.int32)

            def row(r, carry):
                ridx = zi + (k * ck + r)
                av = plsc.load_gather(w1b, [ridx])
                bv = plsc.load_gather(w2b, [ridx])
                for q in range(h // 16):
                    sl = pl.ds(q * 16, 16)
                    rows1[r, sl] = av * rows1[r, sl] + bv * rows2[r, sl]
                return carry
            lax.fori_loop(0, ck, row, 0)
            pltpu.sync_copy(rows1, out_hbm.at[pl.ds(base + k * ck, ck), :])

    return combine


def kernel(hidden_states, gate_w, gate_proj_w, up_proj_w, down_proj_w):
    b, s, h = hidden_states.shape
    e = gate_w.shape[0]
    t = b * s
    x = hidden_states.reshape(t, h)
    (logits, a1, a2, p1, p2, s1, s2, w1, w2, cntf) = _route(x, gate_w)
    cnt = cntf[0].astype(jnp.int32)
    xg = _make_dispatch(t, h, e)(a1[:, 0], a2[:, 0], p1[:, 0], p2[:, 0], x)
    dg = _ffn(cnt, xg.reshape(e * NT, TILE, h),
              gate_proj_w, up_proj_w, down_proj_w)
    final = _make_combine(t, h)(
        dg.reshape(e * C, h), s1[:, 0], s2[:, 0], w1[:, 0], w2[:, 0])
    return final.reshape(b, s, h), logits
